# baseline (speedup 1.0000x reference)
import functools

import jax
import jax.numpy as jnp
from jax import lax
from jax.experimental import pallas as pl
from jax.experimental.pallas import tpu as pltpu
from jax.experimental.pallas import tpu_sc as plsc

N = 10000
E = 320000
D_IN = 128
H = 64

NC = 2
NS = 16
NW = NC * NS
CHUNK = 128
NCH = 80
EPW = NCH * CHUNK
E_PAD = EPW * NW
N_ACC = N + 112
ROWS_PER_SUB = N_ACC // NS
DEG_W = 8
HBM_CH = 26


def _edge_agg(width, with_gather):
    mesh = plsc.VectorSubcoreMesh(core_axis_name="c", subcore_axis_name="s")

    @functools.partial(
        pl.kernel,
        mesh=mesh,
        compiler_params=pltpu.CompilerParams(use_tc_tiling_on_sc=False),
        out_type=jax.ShapeDtypeStruct((NC, N_ACC, width), jnp.float32),
        scratch_types=[
            pltpu.VMEM((NCH, CHUNK), jnp.int32),
            pltpu.VMEM((NCH, CHUNK), jnp.int32),
            pltpu.VMEM((2, CHUNK, width), jnp.float32),
            pltpu.VMEM_SHARED((N_ACC, width), jnp.float32),
            pltpu.VMEM_SHARED((N_ACC, width), jnp.float32)
            if with_gather else pltpu.VMEM((8,), jnp.float32),
            pltpu.SemaphoreType.DMA,
            pltpu.SemaphoreType.DMA,
            pltpu.SemaphoreType.DMA,
            pltpu.SemaphoreType.DMA,
        ],
    )
    def k(table_hbm, src_hbm, dst_hbm, zeros_hbm, out_hbm,
          src_i, dst_i, rows_v, acc, table_s, gs0, gs1, ss0, ss1):
        c = lax.axis_index("c")
        s = lax.axis_index("s")
        wid = c * NS + s
        r0 = s * ROWS_PER_SUB
        pltpu.sync_copy(zeros_hbm.at[pl.ds(r0, ROWS_PER_SUB)],
                        acc.at[pl.ds(r0, ROWS_PER_SUB)])
        pltpu.sync_copy(dst_hbm.at[wid], dst_i)
        if with_gather:
            pltpu.sync_copy(src_hbm.at[wid], src_i)
            pltpu.sync_copy(table_hbm.at[pl.ds(r0, ROWS_PER_SUB)],
                            table_s.at[pl.ds(r0, ROWS_PER_SUB)])
        else:
            pltpu.sync_copy(table_hbm, rows_v.at[0])
        plsc.subcore_barrier()

        gsem = (gs0, gs1)
        ssem = (ss0, ss1)

        def gather(j, b):
            return pltpu.make_async_copy(
                table_s.at[src_i.at[j]], rows_v.at[b], gsem[b])

        def gather_h(j, b):
            return pltpu.make_async_copy(
                table_hbm.at[src_i.at[j]], rows_v.at[b], gsem[b])

        def scatter(j, b):
            return pltpu.make_async_copy(
                rows_v.at[b], acc.at[dst_i.at[j]], ssem[b])

        if with_gather:
            def gstart(j, b):
                @pl.when(j % 4 == 0)
                def _():
                    gather_h(j, b).start()

                @pl.when(j % 4 != 0)
                def _():
                    gather(j, b).start()

            gstart(0, 0)

            def body(j, carry):
                gstart(j + 1, 1)
                gather(j, 0).wait()
                scatter(j, 0).start(add=True)
                scatter(j, 0).wait()

                @pl.when(j + 2 < NCH)
                def _():
                    gstart(j + 2, 0)

                gather(j + 1, 1).wait()
                scatter(j + 1, 1).start(add=True)
                scatter(j + 1, 1).wait()
                return carry

            lax.fori_loop(0, NCH // 2, lambda i, c_: body(2 * i, c_), 0)
        else:
            def sc_deg(j, sm):
                return pltpu.make_async_copy(
                    rows_v.at[0], acc.at[dst_i.at[j]], sm)

            def body(j, carry):
                sc_deg(j, ss0).start(add=True)
                sc_deg(j + 1, ss1).start(add=True)
                sc_deg(j, ss0).wait()
                sc_deg(j + 1, ss1).wait()
                return carry

            lax.fori_loop(0, NCH // 2, lambda i, c_: body(2 * i, c_), 0)

        plsc.subcore_barrier()
        pltpu.sync_copy(acc.at[pl.ds(r0, ROWS_PER_SUB)],
                        out_hbm.at[c, pl.ds(r0, ROWS_PER_SUB)])

    return k


_agg64 = _edge_agg(H, True)
_agg32 = _edge_agg(H // 2, True)
_agg_deg = _edge_agg(DEG_W, False)


def _tc_in(x_ref, wint_ref, bin_ref, w1t_ref, hw1_ref):
    h = jnp.dot(x_ref[...], wint_ref[...], preferred_element_type=jnp.float32)
    h = jnp.maximum(h + bin_ref[...], 0.0)
    hw1_ref[...] = jnp.dot(h, w1t_ref[...], preferred_element_type=jnp.float32)


def _tc_dinv(deg_ref, hw1_ref, dinv_ref, p1_ref):
    deg = deg_ref[0, :, 0:1] + deg_ref[1, :, 0:1] + 1.0
    dinv = lax.rsqrt(deg)
    dinv_ref[...] = dinv
    p1_ref[...] = hw1_ref[...] * dinv


def _tc_mid(q_ref, p_ref, dinv_ref, b_ref, g_ref, be_ref, rm_ref, rv_ref,
            res_ref, wt_ref, h_ref, pn_ref, *, residual):
    dinv = dinv_ref[...]
    agg = dinv * (q_ref[0] + q_ref[1] + p_ref[...]) + b_ref[...]
    bn = (agg - rm_ref[...]) * (g_ref[...] * lax.rsqrt(rv_ref[...] + 1e-5)) \
        + be_ref[...]
    h = jnp.maximum(bn, 0.0)
    if residual:
        h = h + res_ref[...]
    h_ref[...] = h
    pn_ref[...] = dinv * jnp.dot(h, wt_ref[...],
                                 preferred_element_type=jnp.float32)


def _tc_head(q_ref, p_ref, dinv_ref, b_ref, wht_ref, bh_ref, scale_ref,
             out_ref):
    dinv = dinv_ref[...]
    h3 = jnp.maximum(dinv * (q_ref[0] + q_ref[1] + p_ref[...]) + b_ref[...],
                     0.0)
    y = jnp.dot(h3, wht_ref[...], preferred_element_type=jnp.float32)
    out_ref[...] = jnp.tanh(y + bh_ref[...]) * scale_ref[...]


def _dense(body, out_shapes, *args):
    return pl.pallas_call(body, out_shape=out_shapes)(*args)


def kernel(x, edge_index, W_in, b_in, W1, b1, W2, b2, W3, b3, Wh, bh,
           g1, be1, rm1, rv1, g2, be2, rm2, rv2, scale):
    f32 = jnp.float32
    src = edge_index[0]
    dst = edge_index[1]
    pad_e = E_PAD - E
    src_p = jnp.reshape(
        jnp.concatenate([src, jnp.full((pad_e,), N, jnp.int32)]),
        (NW, NCH, CHUNK))
    dst_p = jnp.reshape(
        jnp.concatenate([dst, jnp.full((pad_e,), N, jnp.int32)]),
        (NW, NCH, CHUNK))
    x_p = jnp.concatenate([x, jnp.zeros((N_ACC - N, D_IN), f32)])

    zeros64 = jnp.zeros((N_ACC, H), f32)
    zeros32 = jnp.zeros((N_ACC, H // 2), f32)
    zerosdg = jnp.zeros((N_ACC, DEG_W), f32)
    ones_blk = jnp.ones((CHUNK, DEG_W), f32)

    row = lambda v: jnp.reshape(v, (1, -1))

    deg_parts = _agg_deg(ones_blk, src_p, dst_p, zerosdg)

    hw1 = _dense(_tc_in, jax.ShapeDtypeStruct((N_ACC, H), f32),
                 x_p, W_in.T, row(b_in), W1.T)

    dinv, p1 = _dense(_tc_dinv,
                      (jax.ShapeDtypeStruct((N_ACC, 1), f32),
                       jax.ShapeDtypeStruct((N_ACC, H), f32)),
                      deg_parts, hw1)

    q1 = _agg64(p1, src_p, dst_p, zeros64)
    h1, p2 = _dense(functools.partial(_tc_mid, residual=False),
                    (jax.ShapeDtypeStruct((N_ACC, H), f32),
                     jax.ShapeDtypeStruct((N_ACC, H), f32)),
                    q1, p1, dinv, row(b1), row(g1), row(be1), row(rm1),
                    row(rv1), zeros64, W2.T)

    q2 = _agg64(p2, src_p, dst_p, zeros64)
    _, p3 = _dense(functools.partial(_tc_mid, residual=True),
                   (jax.ShapeDtypeStruct((N_ACC, H), f32),
                    jax.ShapeDtypeStruct((N_ACC, H // 2), f32)),
                   q2, p2, dinv, row(b2), row(g2), row(be2), row(rm2),
                   row(rv2), h1, W3.T)

    q3 = _agg32(p3, src_p, dst_p, zeros32)
    z3 = lambda i: (0, 0, 0)
    z2 = lambda i: (0, 0)
    out = pl.pallas_call(
        _tc_head,
        grid=(1,),
        in_specs=[
            pl.BlockSpec((NC, N, H // 2), z3),
            pl.BlockSpec((N, H // 2), z2),
            pl.BlockSpec((N, 1), z2),
            pl.BlockSpec((1, H // 2), z2),
            pl.BlockSpec((H // 2, 1), z2),
            pl.BlockSpec((1, 1), z2),
            pl.BlockSpec((1, 1), z2),
        ],
        out_specs=pl.BlockSpec((N, 1), z2),
        out_shape=jax.ShapeDtypeStruct((N, 1), f32),
    )(q3, p3, dinv, row(b3), Wh.T, row(bh), jnp.reshape(scale, (1, 1)))
    return out


# speedup vs baseline: 1.3697x; 1.3697x over previous
import functools

import jax
import jax.numpy as jnp
from jax import lax
from jax.experimental import pallas as pl
from jax.experimental.pallas import tpu as pltpu
from jax.experimental.pallas import tpu_sc as plsc

N = 10000
E = 320000
D_IN = 128
H = 64

NC = 2
NS = 16
NW = NC * NS
CHUNK = 128
NCHT = E // CHUNK
NCH_LO = NCHT // NW
NW_HI = NCHT - NCH_LO * NW
RPS = N // NS
DEG_W = 8


def _edge_agg(width, with_gather):
    mesh = plsc.VectorSubcoreMesh(core_axis_name="c", subcore_axis_name="s")

    @functools.partial(
        pl.kernel,
        mesh=mesh,
        compiler_params=pltpu.CompilerParams(use_tc_tiling_on_sc=False),
        out_type=jax.ShapeDtypeStruct((NC, N, width), jnp.float32),
        scratch_types=[
            pltpu.VMEM((NCH_LO + 1, CHUNK), jnp.int32),
            pltpu.VMEM((NCH_LO + 1, CHUNK), jnp.int32),
            pltpu.VMEM((2, CHUNK, width), jnp.float32),
            pltpu.VMEM_SHARED((N, width), jnp.float32),
            pltpu.VMEM_SHARED((N, width), jnp.float32)
            if with_gather else pltpu.VMEM((8,), jnp.float32),
            pltpu.SemaphoreType.DMA,
            pltpu.SemaphoreType.DMA,
            pltpu.SemaphoreType.DMA,
            pltpu.SemaphoreType.DMA,
        ],
    )
    def k(table_hbm, edges_hbm, zeros_hbm, out_hbm,
          src_i, dst_i, rows_v, acc, table_s, gs0, gs1, ss0, ss1):
        c = lax.axis_index("c")
        s = lax.axis_index("s")
        wid = c * NS + s
        has_extra = wid < NW_HI
        cb = wid * NCH_LO + jnp.minimum(wid, NW_HI)
        r0 = s * RPS
        pltpu.sync_copy(zeros_hbm.at[pl.ds(r0, RPS)], acc.at[pl.ds(r0, RPS)])
        pltpu.sync_copy(edges_hbm.at[1, pl.ds(cb, NCH_LO)],
                        dst_i.at[pl.ds(0, NCH_LO)])
        if with_gather:
            pltpu.sync_copy(edges_hbm.at[0, pl.ds(cb, NCH_LO)],
                            src_i.at[pl.ds(0, NCH_LO)])
            pltpu.sync_copy(table_hbm.at[pl.ds(r0, RPS)],
                            table_s.at[pl.ds(r0, RPS)])
        else:
            pltpu.sync_copy(table_hbm, rows_v.at[0])

        @pl.when(has_extra)
        def _():
            pltpu.sync_copy(edges_hbm.at[1, pl.ds(cb + NCH_LO, 1)],
                            dst_i.at[pl.ds(NCH_LO, 1)])
            if with_gather:
                pltpu.sync_copy(edges_hbm.at[0, pl.ds(cb + NCH_LO, 1)],
                                src_i.at[pl.ds(NCH_LO, 1)])

        plsc.subcore_barrier()

        gsem = (gs0, gs1)
        ssem = (ss0, ss1)

        def gather(j, b):
            return pltpu.make_async_copy(
                table_s.at[src_i.at[j]], rows_v.at[b], gsem[b])

        def scatter(j, b):
            return pltpu.make_async_copy(
                rows_v.at[b], acc.at[dst_i.at[j]], ssem[b])

        if with_gather:
            gather(0, 0).start()

            def body(j, carry):
                gather(j + 1, 1).start()
                gather(j, 0).wait()
                scatter(j, 0).start(add=True)
                scatter(j, 0).wait()

                @pl.when(j + 2 < NCH_LO)
                def _():
                    gather(j + 2, 0).start()

                gather(j + 1, 1).wait()
                scatter(j + 1, 1).start(add=True)
                scatter(j + 1, 1).wait()
                return carry

            lax.fori_loop(0, NCH_LO // 2, lambda i, c_: body(2 * i, c_), 0)

            @pl.when(has_extra)
            def _():
                gather(NCH_LO, 0).start()
                gather(NCH_LO, 0).wait()
                scatter(NCH_LO, 0).start(add=True)
                scatter(NCH_LO, 0).wait()
        else:
            def sc_deg(j, sm):
                return pltpu.make_async_copy(
                    rows_v.at[0], acc.at[dst_i.at[j]], sm)

            def body(j, carry):
                sc_deg(j, ss0).start(add=True)
                sc_deg(j + 1, ss1).start(add=True)
                sc_deg(j, ss0).wait()
                sc_deg(j + 1, ss1).wait()
                return carry

            lax.fori_loop(0, NCH_LO // 2, lambda i, c_: body(2 * i, c_), 0)

            @pl.when(has_extra)
            def _():
                sc_deg(NCH_LO, ss0).start(add=True)
                sc_deg(NCH_LO, ss0).wait()

        plsc.subcore_barrier()
        pltpu.sync_copy(acc.at[pl.ds(r0, RPS)],
                        out_hbm.at[c, pl.ds(r0, RPS)])

    return k


_agg64 = _edge_agg(H, True)
_agg32 = _edge_agg(H // 2, True)
_agg_deg = _edge_agg(DEG_W, False)


def _tc_in(x_ref, wint_ref, bin_ref, w1t_ref, hw1_ref):
    h = jnp.dot(x_ref[...], wint_ref[...], preferred_element_type=jnp.float32)
    h = jnp.maximum(h + bin_ref[...], 0.0)
    hw1_ref[...] = jnp.dot(h, w1t_ref[...], preferred_element_type=jnp.float32)


def _tc_dinv(deg_ref, hw1_ref, dinv_ref, p1_ref):
    deg = deg_ref[0, :, 0:1] + deg_ref[1, :, 0:1] + 1.0
    dinv = lax.rsqrt(deg)
    dinv_ref[...] = dinv
    p1_ref[...] = hw1_ref[...] * dinv


def _tc_mid(q_ref, p_ref, dinv_ref, b_ref, g_ref, be_ref, rm_ref, rv_ref,
            res_ref, wt_ref, h_ref, pn_ref, *, residual):
    dinv = dinv_ref[...]
    agg = dinv * (q_ref[0] + q_ref[1] + p_ref[...]) + b_ref[...]
    bn = (agg - rm_ref[...]) * (g_ref[...] * lax.rsqrt(rv_ref[...] + 1e-5)) \
        + be_ref[...]
    h = jnp.maximum(bn, 0.0)
    if residual:
        h = h + res_ref[...]
    h_ref[...] = h
    pn_ref[...] = dinv * jnp.dot(h, wt_ref[...],
                                 preferred_element_type=jnp.float32)


def _tc_head(q_ref, p_ref, dinv_ref, b_ref, wht_ref, bh_ref, scale_ref,
             out_ref):
    dinv = dinv_ref[...]
    h3 = jnp.maximum(dinv * (q_ref[0] + q_ref[1] + p_ref[...]) + b_ref[...],
                     0.0)
    y = jnp.dot(h3, wht_ref[...], preferred_element_type=jnp.float32)
    out_ref[...] = jnp.tanh(y + bh_ref[...]) * scale_ref[...]


def _dense(body, out_shapes, *args):
    return pl.pallas_call(body, out_shape=out_shapes)(*args)


def kernel(x, edge_index, W_in, b_in, W1, b1, W2, b2, W3, b3, Wh, bh,
           g1, be1, rm1, rv1, g2, be2, rm2, rv2, scale):
    f32 = jnp.float32
    edges = jnp.reshape(edge_index, (2, NCHT, CHUNK))

    zeros64 = jnp.zeros((N, H), f32)
    zeros32 = jnp.zeros((N, H // 2), f32)
    zerosdg = jnp.zeros((N, DEG_W), f32)
    ones_blk = jnp.ones((CHUNK, DEG_W), f32)

    row = lambda v: jnp.reshape(v, (1, -1))

    deg_parts = _agg_deg(ones_blk, edges, zerosdg)

    hw1 = _dense(_tc_in, jax.ShapeDtypeStruct((N, H), f32),
                 x, W_in.T, row(b_in), W1.T)

    dinv, p1 = _dense(_tc_dinv,
                      (jax.ShapeDtypeStruct((N, 1), f32),
                       jax.ShapeDtypeStruct((N, H), f32)),
                      deg_parts, hw1)

    q1 = _agg64(p1, edges, zeros64)
    h1, p2 = _dense(functools.partial(_tc_mid, residual=False),
                    (jax.ShapeDtypeStruct((N, H), f32),
                     jax.ShapeDtypeStruct((N, H), f32)),
                    q1, p1, dinv, row(b1), row(g1), row(be1), row(rm1),
                    row(rv1), zeros64, W2.T)

    q2 = _agg64(p2, edges, zeros64)
    _, p3 = _dense(functools.partial(_tc_mid, residual=True),
                   (jax.ShapeDtypeStruct((N, H), f32),
                    jax.ShapeDtypeStruct((N, H // 2), f32)),
                   q2, p2, dinv, row(b2), row(g2), row(be2), row(rm2),
                   row(rv2), h1, W3.T)

    q3 = _agg32(p3, edges, zeros32)
    out = _dense(_tc_head, jax.ShapeDtypeStruct((N, 1), f32),
                 q3, p3, dinv, row(b3), Wh.T, row(bh),
                 jnp.reshape(scale, (1, 1)))
    return out
